# baseline (speedup 1.0000x reference)
import functools

import jax
import jax.numpy as jnp
from jax import lax
from jax.experimental import pallas as pl
from jax.experimental.pallas import tpu as pltpu, tpu_sc as plsc

N = 10000
E = 320000
D = 128
HID = 16
NPAD = 10112
NC, NS = 2, 16
NW = NC * NS
EPW = E // NW
CH = 80
NCHUNK = EPW // CH
ROWS_PER_SUB = NPAD // NS

_mesh = plsc.VectorSubcoreMesh(core_axis_name="c", subcore_axis_name="s")


@functools.partial(
    pl.kernel,
    out_type=(jax.ShapeDtypeStruct((NC * NPAD, D), jnp.float32),
              jax.ShapeDtypeStruct((NC * NPAD, HID), jnp.float32)),
    mesh=_mesh,
    scratch_types=[
        pltpu.VMEM((NCHUNK, CH), jnp.int32),
        pltpu.VMEM((NCHUNK, CH), jnp.int32),
        pltpu.VMEM((CH, D), jnp.float32),
        pltpu.VMEM((CH, HID), jnp.float32),
        pltpu.VMEM_SHARED((NPAD, D), jnp.float32),
        pltpu.VMEM_SHARED((NPAD, HID), jnp.float32),
        pltpu.SemaphoreType.DMA,
    ],
    compiler_params=pltpu.CompilerParams(use_tc_tiling_on_sc=False),
)
def _sc_pass1(src_hbm, dst_hbm, table_hbm, zeros_d_hbm, zeros_h_hbm, ones_hbm,
              sum_hbm, deg_hbm, sidx, didx, rows, ones, acc, dacc, sem):
    c = lax.axis_index("c")
    s = lax.axis_index("s")
    wid = c * NS + s
    sl = pl.ds(s * ROWS_PER_SUB, ROWS_PER_SUB)
    pltpu.sync_copy(zeros_d_hbm, acc.at[sl])
    pltpu.sync_copy(zeros_h_hbm, dacc.at[sl])
    pltpu.sync_copy(src_hbm.at[wid], sidx)
    pltpu.sync_copy(dst_hbm.at[wid], didx)
    pltpu.sync_copy(ones_hbm, ones)
    plsc.subcore_barrier()

    def body(i, carry):
        pltpu.async_copy(table_hbm.at[sidx.at[i]], rows, sem).wait()
        pltpu.sync_copy(rows, acc.at[didx.at[i]], add=True)
        pltpu.sync_copy(ones, dacc.at[didx.at[i]], add=True)
        return carry

    lax.fori_loop(0, NCHUNK, body, 0)
    plsc.subcore_barrier()
    pltpu.sync_copy(acc.at[sl], sum_hbm.at[pl.ds(c * NPAD + s * ROWS_PER_SUB,
                                                 ROWS_PER_SUB)])
    pltpu.sync_copy(dacc.at[sl], deg_hbm.at[pl.ds(c * NPAD + s * ROWS_PER_SUB,
                                                  ROWS_PER_SUB)])


@functools.partial(
    pl.kernel,
    out_type=jax.ShapeDtypeStruct((NC * NPAD, HID), jnp.float32),
    mesh=_mesh,
    scratch_types=[
        pltpu.VMEM((NCHUNK, CH), jnp.int32),
        pltpu.VMEM((NCHUNK, CH), jnp.int32),
        pltpu.VMEM((CH, HID), jnp.float32),
        pltpu.VMEM_SHARED((NPAD, HID), jnp.float32),
        pltpu.VMEM_SHARED((NPAD, HID), jnp.float32),
        pltpu.SemaphoreType.DMA,
    ],
    compiler_params=pltpu.CompilerParams(use_tc_tiling_on_sc=False),
)
def _sc_pass2(src_hbm, dst_hbm, table_hbm, zeros_hbm, out_hbm,
              sidx, didx, rows, tbl, acc, sem):
    c = lax.axis_index("c")
    s = lax.axis_index("s")
    wid = c * NS + s
    sl = pl.ds(s * ROWS_PER_SUB, ROWS_PER_SUB)
    pltpu.sync_copy(table_hbm.at[sl], tbl.at[sl])
    pltpu.sync_copy(zeros_hbm, acc.at[sl])
    pltpu.sync_copy(src_hbm.at[wid], sidx)
    pltpu.sync_copy(dst_hbm.at[wid], didx)
    plsc.subcore_barrier()

    def body(i, carry):
        pltpu.async_copy(tbl.at[sidx.at[i]], rows, sem).wait()
        pltpu.sync_copy(rows, acc.at[didx.at[i]], add=True)
        return carry

    lax.fori_loop(0, NCHUNK, body, 0)
    plsc.subcore_barrier()
    pltpu.sync_copy(acc.at[sl], out_hbm.at[pl.ds(c * NPAD + s * ROWS_PER_SUB,
                                                 ROWS_PER_SUB)])


def _elu(x):
    p = x * (1.0 + x * (0.5 + x * (1.0 / 6.0 + x * (1.0 / 24.0 + x * (
        1.0 / 120.0 + x * (1.0 / 720.0 + x * (1.0 / 5040.0 + x / 40320.0)))))))
    em1 = jnp.where(x > -0.5, p, jnp.exp(x) - 1.0)
    return jnp.where(x > 0, x, em1)


def _deg_tile(degc_hid, width):
    return jnp.concatenate([degc_hid] * (width // HID), axis=1)


def _tca(sum_ref, dacc_ref, n_ref, w1s_ref, w1n_ref, b1_ref, h_ref, degc_ref):
    summed = sum_ref[0] + sum_ref[1]
    degc = jnp.clip(dacc_ref[0] + dacc_ref[1], 1.0, None)
    mean = summed / _deg_tile(degc, D)
    h = (jnp.dot(n_ref[...], w1s_ref[...], preferred_element_type=jnp.float32)
         + jnp.dot(mean, w1n_ref[...], preferred_element_type=jnp.float32)
         + b1_ref[...])
    h_ref[...] = _elu(h)
    degc_ref[...] = degc


def _bf16(x):
    return x.astype(jnp.bfloat16).astype(jnp.float32)


def _vecmat(y_row, w, eye, mxu):
    y_col = jnp.sum(eye * y_row, axis=1, keepdims=True)
    if mxu:
        y_col = _bf16(y_col)
        w = _bf16(w)
    return jnp.sum(y_col * w, axis=0, keepdims=True)


def _tcb(acc2_ref, h_ref, degc_ref, w2s_ref, w2n_ref, b2_ref,
         gg_ref, gb_ref, wg_ref, bg_ref, fg_ref, fb_ref,
         wf1_ref, bf1_ref, wf2_ref, bf2_ref, wf3_ref, bf3_ref, eye_ref,
         out_ref):
    inv_sqrt = jnp.sqrt(1.0 + 1e-5)
    summed2 = acc2_ref[0:NPAD, :] + acc2_ref[NPAD:, :]
    mean2 = summed2 / degc_ref[...]
    h2 = (jnp.dot(h_ref[...], w2s_ref[...], preferred_element_type=jnp.float32)
          + jnp.dot(mean2, w2n_ref[...], preferred_element_type=jnp.float32)
          + b2_ref[...])
    hn = h2[0:N, :]
    hbn = hn / inv_sqrt * gg_ref[...] + gb_ref[...]
    gate = jnp.sum(hbn * wg_ref[...], axis=1, keepdims=True) + bg_ref[...]
    m = jnp.max(gate)
    ex = jnp.exp(gate - m)
    attn = ex / jnp.sum(ex)
    h1 = jnp.sum(attn * hn, axis=0, keepdims=True)
    hmax = jnp.max(hn, axis=0, keepdims=True)
    hc = _elu(jnp.concatenate([h1, hmax], axis=1))
    y = hc / inv_sqrt * fg_ref[...] + fb_ref[...]
    eye = eye_ref[...]
    y = jnp.maximum(_vecmat(y, wf1_ref[...], eye, True) + bf1_ref[...], 0.0)
    y = jnp.maximum(_vecmat(y, wf2_ref[...], eye, True) + bf2_ref[...], 0.0)
    out_ref[...] = _vecmat(y, wf3_ref[...], eye, False) + bf3_ref[...]


def kernel(n, edge_index, e, p, W1s, W1n, b1, W2s, W2n, b2,
           gn_gamma, gn_beta, Wg, bg, f_gamma, f_beta,
           Wf1, bf1, Wf2, bf2, Wf3, bf3):
    src = edge_index[0].reshape(NW, NCHUNK, CH)
    dst = edge_index[1].reshape(NW, NCHUNK, CH)
    npad = jnp.pad(n, ((0, NPAD - N), (0, 0)))

    sum1, dacc = _sc_pass1(src, dst, npad,
                           jnp.zeros((ROWS_PER_SUB, D), jnp.float32),
                           jnp.zeros((ROWS_PER_SUB, HID), jnp.float32),
                           jnp.ones((CH, HID), jnp.float32))

    nb = 8
    rb = NPAD // nb
    h, degc = pl.pallas_call(
        _tca,
        grid=(nb,),
        in_specs=[
            pl.BlockSpec((NC, rb, D), lambda i: (0, i, 0)),
            pl.BlockSpec((NC, rb, HID), lambda i: (0, i, 0)),
            pl.BlockSpec((rb, D), lambda i: (i, 0)),
            pl.BlockSpec((D, HID), lambda i: (0, 0)),
            pl.BlockSpec((D, HID), lambda i: (0, 0)),
            pl.BlockSpec((1, HID), lambda i: (0, 0)),
        ],
        out_specs=[
            pl.BlockSpec((rb, HID), lambda i: (i, 0)),
            pl.BlockSpec((rb, HID), lambda i: (i, 0)),
        ],
        out_shape=(jax.ShapeDtypeStruct((NPAD, HID), jnp.float32),
                   jax.ShapeDtypeStruct((NPAD, HID), jnp.float32)),
    )(sum1.reshape(NC, NPAD, D), dacc.reshape(NC, NPAD, HID),
      npad, W1s, W1n, b1.reshape(1, HID))

    acc2 = _sc_pass2(src, dst, h, jnp.zeros((ROWS_PER_SUB, HID), jnp.float32))

    out = pl.pallas_call(
        _tcb,
        out_shape=jax.ShapeDtypeStruct((1, 1), jnp.float32),
    )(acc2, h, degc, W2s, W2n, b2.reshape(1, HID),
      gn_gamma.reshape(1, HID), gn_beta.reshape(1, HID),
      Wg.reshape(1, HID), bg.reshape(1, 1),
      f_gamma.reshape(1, 32), f_beta.reshape(1, 32),
      Wf1, bf1.reshape(1, 32), Wf2, bf2.reshape(1, 32),
      Wf3, bf3.reshape(1, 1), jnp.eye(32, dtype=jnp.float32))
    return out


# speedup vs baseline: 1.1186x; 1.1186x over previous
import functools

import jax
import jax.numpy as jnp
from jax import lax
from jax.experimental import pallas as pl
from jax.experimental.pallas import tpu as pltpu, tpu_sc as plsc

N = 10000
E = 320000
D = 128
HID = 16
NPAD = 10112
NC, NS = 2, 16
NW = NC * NS
EPW = E // NW
CH = 125
NCHUNK = EPW // CH
ROWS_PER_SUB = NPAD // NS

_mesh = plsc.VectorSubcoreMesh(core_axis_name="c", subcore_axis_name="s")


@functools.partial(
    pl.kernel,
    out_type=(jax.ShapeDtypeStruct((NC * NPAD, D), jnp.float32),
              jax.ShapeDtypeStruct((NC * NPAD, HID), jnp.float32)),
    mesh=_mesh,
    scratch_types=[
        pltpu.VMEM((NCHUNK, CH), jnp.int32),
        pltpu.VMEM((NCHUNK, CH), jnp.int32),
        pltpu.VMEM((CH, D), jnp.float32),
        pltpu.VMEM((CH, HID), jnp.float32),
        pltpu.VMEM_SHARED((NPAD, D), jnp.float32),
        pltpu.VMEM_SHARED((NPAD, HID), jnp.float32),
        pltpu.SemaphoreType.DMA,
    ],
    compiler_params=pltpu.CompilerParams(use_tc_tiling_on_sc=False),
)
def _sc_pass1(src_hbm, dst_hbm, table_hbm, zeros_d_hbm, zeros_h_hbm, ones_hbm,
              sum_hbm, deg_hbm, sidx, didx, rows, ones, acc, dacc, sem):
    c = lax.axis_index("c")
    s = lax.axis_index("s")
    wid = c * NS + s
    sl = pl.ds(s * ROWS_PER_SUB, ROWS_PER_SUB)
    pltpu.sync_copy(zeros_d_hbm, acc.at[sl])
    pltpu.sync_copy(zeros_h_hbm, dacc.at[sl])
    pltpu.sync_copy(src_hbm.at[wid], sidx)
    pltpu.sync_copy(dst_hbm.at[wid], didx)
    pltpu.sync_copy(ones_hbm, ones)
    plsc.subcore_barrier()

    def body(i, carry):
        pltpu.async_copy(table_hbm.at[sidx.at[i]], rows, sem).wait()
        pltpu.sync_copy(rows, acc.at[didx.at[i]], add=True)
        pltpu.sync_copy(ones, dacc.at[didx.at[i]], add=True)
        return carry

    lax.fori_loop(0, NCHUNK, body, 0)
    plsc.subcore_barrier()
    pltpu.sync_copy(acc.at[sl], sum_hbm.at[pl.ds(c * NPAD + s * ROWS_PER_SUB,
                                                 ROWS_PER_SUB)])
    pltpu.sync_copy(dacc.at[sl], deg_hbm.at[pl.ds(c * NPAD + s * ROWS_PER_SUB,
                                                  ROWS_PER_SUB)])


@functools.partial(
    pl.kernel,
    out_type=jax.ShapeDtypeStruct((NC * NPAD, HID), jnp.float32),
    mesh=_mesh,
    scratch_types=[
        pltpu.VMEM((NCHUNK, CH), jnp.int32),
        pltpu.VMEM((NCHUNK, CH), jnp.int32),
        pltpu.VMEM((CH, HID), jnp.float32),
        pltpu.VMEM_SHARED((NPAD, HID), jnp.float32),
        pltpu.VMEM_SHARED((NPAD, HID), jnp.float32),
        pltpu.SemaphoreType.DMA,
    ],
    compiler_params=pltpu.CompilerParams(use_tc_tiling_on_sc=False),
)
def _sc_pass2(src_hbm, dst_hbm, table_hbm, zeros_hbm, out_hbm,
              sidx, didx, rows, tbl, acc, sem):
    c = lax.axis_index("c")
    s = lax.axis_index("s")
    wid = c * NS + s
    sl = pl.ds(s * ROWS_PER_SUB, ROWS_PER_SUB)
    pltpu.sync_copy(table_hbm.at[sl], tbl.at[sl])
    pltpu.sync_copy(zeros_hbm, acc.at[sl])
    pltpu.sync_copy(src_hbm.at[wid], sidx)
    pltpu.sync_copy(dst_hbm.at[wid], didx)
    plsc.subcore_barrier()

    def body(i, carry):
        pltpu.async_copy(tbl.at[sidx.at[i]], rows, sem).wait()
        pltpu.sync_copy(rows, acc.at[didx.at[i]], add=True)
        return carry

    lax.fori_loop(0, NCHUNK, body, 0)
    plsc.subcore_barrier()
    pltpu.sync_copy(acc.at[sl], out_hbm.at[pl.ds(c * NPAD + s * ROWS_PER_SUB,
                                                 ROWS_PER_SUB)])


def _elu(x):
    p = x * (1.0 + x * (0.5 + x * (1.0 / 6.0 + x * (1.0 / 24.0 + x * (
        1.0 / 120.0 + x * (1.0 / 720.0 + x * (1.0 / 5040.0 + x / 40320.0)))))))
    em1 = jnp.where(x > -0.5, p, jnp.exp(x) - 1.0)
    return jnp.where(x > 0, x, em1)


def _deg_tile(degc_hid, width):
    return jnp.concatenate([degc_hid] * (width // HID), axis=1)


def _tca(sum_ref, dacc_ref, n_ref, w1s_ref, w1n_ref, b1_ref, h_ref, degc_ref):
    summed = sum_ref[0] + sum_ref[1]
    degc = jnp.clip(dacc_ref[0] + dacc_ref[1], 1.0, None)
    mean = summed / _deg_tile(degc, D)
    h = (jnp.dot(n_ref[...], w1s_ref[...], preferred_element_type=jnp.float32)
         + jnp.dot(mean, w1n_ref[...], preferred_element_type=jnp.float32)
         + b1_ref[...])
    h_ref[...] = _elu(h)
    degc_ref[...] = degc


def _bf16(x):
    return x.astype(jnp.bfloat16).astype(jnp.float32)


def _vecmat(y_row, w, eye, mxu):
    y_col = jnp.sum(eye * y_row, axis=1, keepdims=True)
    if mxu:
        y_col = _bf16(y_col)
        w = _bf16(w)
    return jnp.sum(y_col * w, axis=0, keepdims=True)


def _tcb(acc2_ref, h_ref, degc_ref, w2s_ref, w2n_ref, b2_ref,
         gg_ref, gb_ref, wg_ref, bg_ref, fg_ref, fb_ref,
         wf1_ref, bf1_ref, wf2_ref, bf2_ref, wf3_ref, bf3_ref, eye_ref,
         out_ref):
    inv_sqrt = jnp.sqrt(1.0 + 1e-5)
    summed2 = acc2_ref[0:NPAD, :] + acc2_ref[NPAD:, :]
    mean2 = summed2 / degc_ref[...]
    h2 = (jnp.dot(h_ref[...], w2s_ref[...], preferred_element_type=jnp.float32)
          + jnp.dot(mean2, w2n_ref[...], preferred_element_type=jnp.float32)
          + b2_ref[...])
    hn = h2[0:N, :]
    hbn = hn / inv_sqrt * gg_ref[...] + gb_ref[...]
    gate = jnp.sum(hbn * wg_ref[...], axis=1, keepdims=True) + bg_ref[...]
    m = jnp.max(gate)
    ex = jnp.exp(gate - m)
    attn = ex / jnp.sum(ex)
    h1 = jnp.sum(attn * hn, axis=0, keepdims=True)
    hmax = jnp.max(hn, axis=0, keepdims=True)
    hc = _elu(jnp.concatenate([h1, hmax], axis=1))
    y = hc / inv_sqrt * fg_ref[...] + fb_ref[...]
    eye = eye_ref[...]
    y = jnp.maximum(_vecmat(y, wf1_ref[...], eye, True) + bf1_ref[...], 0.0)
    y = jnp.maximum(_vecmat(y, wf2_ref[...], eye, True) + bf2_ref[...], 0.0)
    out_ref[...] = _vecmat(y, wf3_ref[...], eye, False) + bf3_ref[...]


def kernel(n, edge_index, e, p, W1s, W1n, b1, W2s, W2n, b2,
           gn_gamma, gn_beta, Wg, bg, f_gamma, f_beta,
           Wf1, bf1, Wf2, bf2, Wf3, bf3):
    src = edge_index[0].reshape(NW, NCHUNK, CH)
    dst = edge_index[1].reshape(NW, NCHUNK, CH)
    npad = jnp.pad(n, ((0, NPAD - N), (0, 0)))

    sum1, dacc = _sc_pass1(src, dst, npad,
                           jnp.zeros((ROWS_PER_SUB, D), jnp.float32),
                           jnp.zeros((ROWS_PER_SUB, HID), jnp.float32),
                           jnp.ones((CH, HID), jnp.float32))

    nb = 8
    rb = NPAD // nb
    h, degc = pl.pallas_call(
        _tca,
        grid=(nb,),
        in_specs=[
            pl.BlockSpec((NC, rb, D), lambda i: (0, i, 0)),
            pl.BlockSpec((NC, rb, HID), lambda i: (0, i, 0)),
            pl.BlockSpec((rb, D), lambda i: (i, 0)),
            pl.BlockSpec((D, HID), lambda i: (0, 0)),
            pl.BlockSpec((D, HID), lambda i: (0, 0)),
            pl.BlockSpec((1, HID), lambda i: (0, 0)),
        ],
        out_specs=[
            pl.BlockSpec((rb, HID), lambda i: (i, 0)),
            pl.BlockSpec((rb, HID), lambda i: (i, 0)),
        ],
        out_shape=(jax.ShapeDtypeStruct((NPAD, HID), jnp.float32),
                   jax.ShapeDtypeStruct((NPAD, HID), jnp.float32)),
    )(sum1.reshape(NC, NPAD, D), dacc.reshape(NC, NPAD, HID),
      npad, W1s, W1n, b1.reshape(1, HID))

    acc2 = _sc_pass2(src, dst, h, jnp.zeros((ROWS_PER_SUB, HID), jnp.float32))

    out = pl.pallas_call(
        _tcb,
        out_shape=jax.ShapeDtypeStruct((1, 1), jnp.float32),
    )(acc2, h, degc, W2s, W2n, b2.reshape(1, HID),
      gn_gamma.reshape(1, HID), gn_beta.reshape(1, HID),
      Wg.reshape(1, HID), bg.reshape(1, 1),
      f_gamma.reshape(1, 32), f_beta.reshape(1, 32),
      Wf1, bf1.reshape(1, 32), Wf2, bf2.reshape(1, 32),
      Wf3, bf3.reshape(1, 1), jnp.eye(32, dtype=jnp.float32))
    return out
